# baseline (speedup 1.0000x reference)
import jax
import jax.numpy as jnp
from jax.experimental import pallas as pl
from jax.experimental.pallas import tpu as pltpu

_MEM = 100000
_D = 256
_NC = 10
_B = 1000
_NB = _MEM // (_NC * _B)
_CHUNK = _MEM // _NC


def _score(keys, q, qn2):
    dots = jnp.sum(keys * q, axis=1)
    kn2 = jnp.sum(keys * keys, axis=1)
    return dots * jnp.abs(dots) / jnp.maximum(qn2 * kn2, 1e-16)


def _chunk_update(i, c, q, qn2, ptr, keys, best_v, best_b):
    base = c * _CHUNK + i * _B
    m = jnp.max(_score(keys, q, qn2))
    better = (base + _B <= ptr) & (
        (m > best_v[0]) | ((m == best_v[0]) & (base < best_b[0])))

    @pl.when(better)
    def _():
        best_v[0] = m
        best_b[0] = base


def _scan_body(ptr_ref, q_ref, *rest):
    mrefs = rest[:_NC]
    blk_ref, best_v, best_b = rest[_NC], rest[_NC + 1], rest[_NC + 2]
    i = pl.program_id(0)

    @pl.when(i == 0)
    def _():
        best_v[0] = -jnp.inf
        best_b[0] = 0

    q = q_ref[...]
    qn2 = jnp.sum(q * q)
    ptr = ptr_ref[0]
    for c, mref in enumerate(mrefs):
        _chunk_update(i, c, q, qn2, ptr, mref[...], best_v, best_b)

    @pl.when(i == pl.num_programs(0) - 1)
    def _():
        blk_ref[0] = best_b[0]


def _masked_best(keys, q, qn2, base, ptr):
    s = _score(keys, q, qn2)
    gidx = base + jax.lax.iota(jnp.int32, _B)
    s = jnp.where(gidx < ptr, s, -jnp.inf)
    m = jnp.max(s)
    gi = base + jnp.argmax(s).astype(jnp.int32)
    return m, gi


def _pick_body(s_ref, q_ref, k1_ref, k2_ref, mem_ref, out_ref, vsem):
    b1 = s_ref[0]
    pb = s_ref[1]
    ptr = s_ref[2]
    q = q_ref[...]
    qn2 = jnp.sum(q * q)
    m1, g1 = _masked_best(k1_ref[...], q, qn2, b1, ptr)
    m2, g2 = _masked_best(k2_ref[...], q, qn2, pb, ptr)
    take2 = (m2 > m1) | ((m2 == m1) & (g2 < g1))
    gi = jnp.where(take2, g2, g1)
    copy = pltpu.make_async_copy(
        mem_ref.at[pl.ds(gi, 1), pl.ds(_D, _D)], out_ref, vsem)
    copy.start()
    copy.wait()


def kernel(query, memory, pointer):
    q2 = query.reshape(1, _D)
    ptr = jnp.asarray(pointer, jnp.int32).reshape(1)

    def _mspec(c):
        nblk = _CHUNK // _B
        return pl.BlockSpec((_B, _D), lambda i, p, c=c: (c * nblk + i, 0))

    blk = pl.pallas_call(
        _scan_body,
        grid_spec=pltpu.PrefetchScalarGridSpec(
            num_scalar_prefetch=1,
            grid=(_NB,),
            in_specs=[pl.BlockSpec((1, _D), lambda i, p: (0, 0))]
            + [_mspec(c) for c in range(_NC)],
            out_specs=pl.BlockSpec(memory_space=pltpu.SMEM),
            scratch_shapes=[
                pltpu.SMEM((1,), jnp.float32),
                pltpu.SMEM((1,), jnp.int32),
            ],
        ),
        out_shape=jax.ShapeDtypeStruct((1,), jnp.int32),
    )(ptr, q2, *([memory] * _NC))

    pb = jnp.minimum((ptr // _B) * _B, _MEM - _B)
    sarg = jnp.concatenate([blk, pb, ptr])
    row = pl.pallas_call(
        _pick_body,
        grid_spec=pltpu.PrefetchScalarGridSpec(
            num_scalar_prefetch=1,
            grid=(1,),
            in_specs=[
                pl.BlockSpec((1, _D), lambda i, s: (0, 0)),
                pl.BlockSpec((_B, _D), lambda i, s: (s[0] // _B, 0)),
                pl.BlockSpec((_B, _D), lambda i, s: (s[1] // _B, 0)),
                pl.BlockSpec(memory_space=pl.ANY),
            ],
            out_specs=pl.BlockSpec((1, _D), lambda i, s: (0, 0)),
            scratch_shapes=[pltpu.SemaphoreType.DMA],
        ),
        out_shape=jax.ShapeDtypeStruct((1, _D), jnp.float32),
    )(sarg, q2, memory, memory, memory)

    return row.reshape(_D)


# speedup vs baseline: 1.0213x; 1.0213x over previous
import jax
import jax.numpy as jnp
from jax.experimental import pallas as pl
from jax.experimental.pallas import tpu as pltpu

_MEM = 100000
_D = 256
_NC = 10
_B = 1000
_NB = _MEM // (_NC * _B)
_CHUNK = _MEM // _NC


def _score(keys, q, qn2):
    dots = jnp.sum(keys * q, axis=1)
    kn2 = jnp.sum(keys * keys, axis=1)
    return dots * jnp.abs(dots) / jnp.maximum(qn2 * kn2, 1e-16)


def _masked_best(keys, q, qn2, base, ptr):
    s = _score(keys, q, qn2)
    gidx = base + jax.lax.iota(jnp.int32, _B)
    s = jnp.where(gidx < ptr, s, -jnp.inf)
    m = jnp.max(s)
    gi = base + jnp.argmax(s).astype(jnp.int32)
    return m, gi


def _chunk_update(i, c, q, qn2, ptr, keys, best_v, best_b):
    base = c * _CHUNK + i * _B
    m = jnp.max(_score(keys, q, qn2))
    better = (base + _B <= ptr) & (
        (m > best_v[0]) | ((m == best_v[0]) & (base < best_b[0])))

    @pl.when(better)
    def _():
        best_v[0] = m
        best_b[0] = base


def _scan_body(ptr_ref, q_ref, *rest):
    mrefs = rest[:_NC]
    mem_any = rest[_NC]
    idx_ref = rest[_NC + 1]
    best_v, best_b, blk1, blk2, sem1, sem2 = rest[_NC + 2:]
    i = pl.program_id(0)

    @pl.when(i == 0)
    def _():
        best_v[0] = -jnp.inf
        best_b[0] = 0

    q = q_ref[...]
    qn2 = jnp.sum(q * q)
    ptr = ptr_ref[0]
    for c, mref in enumerate(mrefs):
        _chunk_update(i, c, q, qn2, ptr, mref[...], best_v, best_b)

    @pl.when(i == pl.num_programs(0) - 1)
    def _():
        b1 = pl.multiple_of(best_b[0], 8)
        pb = pl.multiple_of(jnp.minimum((ptr // _B) * _B, _MEM - _B), 8)
        c1 = pltpu.make_async_copy(
            mem_any.at[pl.ds(b1, _B), pl.ds(0, _D)], blk1, sem1)
        c2 = pltpu.make_async_copy(
            mem_any.at[pl.ds(pb, _B), pl.ds(0, _D)], blk2, sem2)
        c1.start()
        c2.start()
        c1.wait()
        c2.wait()
        m1, g1 = _masked_best(blk1[...], q, qn2, b1, ptr)
        m2, g2 = _masked_best(blk2[...], q, qn2, pb, ptr)
        take2 = (m2 > m1) | ((m2 == m1) & (g2 < g1))
        gi = jnp.where(take2, g2, g1)
        idx_ref[0] = gi


def _gather_body(idx_ref, mem_ref, out_ref):
    sub = idx_ref[0] - 8 * (idx_ref[0] // 8)
    out_ref[...] = mem_ref[pl.ds(sub, 1), _D:]


def kernel(query, memory, pointer):
    q2 = query.reshape(1, _D)
    ptr = jnp.asarray(pointer, jnp.int32).reshape(1)

    def _mspec(c):
        nblk = _CHUNK // _B
        return pl.BlockSpec((_B, _D), lambda i, p, c=c: (c * nblk + i, 0))

    idx = pl.pallas_call(
        _scan_body,
        grid_spec=pltpu.PrefetchScalarGridSpec(
            num_scalar_prefetch=1,
            grid=(_NB,),
            in_specs=[pl.BlockSpec((1, _D), lambda i, p: (0, 0))]
            + [_mspec(c) for c in range(_NC)]
            + [pl.BlockSpec(memory_space=pl.ANY)],
            out_specs=pl.BlockSpec(memory_space=pltpu.SMEM),
            scratch_shapes=[
                pltpu.SMEM((1,), jnp.float32),
                pltpu.SMEM((1,), jnp.int32),
                pltpu.VMEM((_B, _D), jnp.float32),
                pltpu.VMEM((_B, _D), jnp.float32),
                pltpu.SemaphoreType.DMA,
                pltpu.SemaphoreType.DMA,
            ],
        ),
        out_shape=jax.ShapeDtypeStruct((1,), jnp.int32),
    )(ptr, q2, *([memory] * _NC), memory)

    row = pl.pallas_call(
        _gather_body,
        grid_spec=pltpu.PrefetchScalarGridSpec(
            num_scalar_prefetch=1,
            grid=(1,),
            in_specs=[
                pl.BlockSpec((8, 2 * _D), lambda i, s: (s[0] // 8, 0)),
            ],
            out_specs=pl.BlockSpec((1, _D), lambda i, s: (0, 0)),
        ),
        out_shape=jax.ShapeDtypeStruct((1, _D), jnp.float32),
    )(idx, memory)

    return row.reshape(_D)
